# Initial kernel scaffold; baseline (speedup 1.0000x reference)
import jax
import jax.numpy as jnp
from jax.experimental import pallas as pl


def kernel(inputs, tok_table, pos_table):
    raise NotImplementedError("write your pallas kernel here")


# speedup vs baseline: 2.7587x; 2.7587x over previous
import functools

import jax
import jax.numpy as jnp
from jax import lax
from jax.experimental import pallas as pl
from jax.experimental.pallas import tpu as pltpu
from jax.experimental.pallas import tpu_sc as plsc

_LANES = 16
_C = 512
_ISUB = 128
_NSUB = _C // _ISUB


def _make_kernel(L, B, E, V, P):
    N = L * B
    n_chunks = N // _C
    info = plsc.get_sparse_core_info()
    NC, NS = info.num_cores, info.num_subcores
    NW = NC * NS
    tpw = n_chunks // NW
    chunks_per_l = B // _C
    assert B % _C == 0 and N % (_C * NW) == 0 and tpw % 2 == 0
    assert E % _LANES == 0
    ev = E // _LANES
    scale = float(E) ** 0.5

    mesh = plsc.VectorSubcoreMesh(core_axis_name="c", subcore_axis_name="s")

    def gather_start(t, inputs_hbm, tok_hbm, idx_v, rows_v, sem):
        pltpu.sync_copy(inputs_hbm.at[pl.ds(t * _NSUB, _NSUB)], idx_v)
        for j in range(_NSUB):
            pltpu.async_copy(
                tok_hbm.at[idx_v.at[j]],
                rows_v.at[pl.ds(j * _ISUB, _ISUB)],
                sem,
            )

    def gather_wait(tok_hbm, idx_v, rows_v, sem):
        for j in range(_NSUB):
            pltpu.make_async_copy(
                tok_hbm.at[idx_v.at[j]],
                rows_v.at[pl.ds(j * _ISUB, _ISUB)],
                sem,
            ).wait()

    def process(t, tok_hbm, pos_hbm, out_hbm, idx_v, rows_v, pos_v, sem):
        l = t // chunks_per_l
        pltpu.sync_copy(pos_hbm.at[pl.ds(l, 1)], pos_v)
        pvs = [pos_v[0, pl.ds(j * _LANES, _LANES)] for j in range(ev)]
        gather_wait(tok_hbm, idx_v, rows_v, sem)

        def rbody(r, _):
            for j in range(ev):
                sl = pl.ds(j * _LANES, _LANES)
                rows_v[r, sl] = rows_v[r, sl] * scale + pvs[j]
            return _

        lax.fori_loop(0, _C, rbody, 0, unroll=2)
        pltpu.sync_copy(rows_v, out_hbm.at[pl.ds(t * _C, _C)])

    @functools.partial(
        pl.kernel,
        mesh=mesh,
        out_type=jax.ShapeDtypeStruct((N, E), jnp.float32),
        compiler_params=pltpu.CompilerParams(use_tc_tiling_on_sc=False),
        scratch_types=[
            pltpu.VMEM((_NSUB, _ISUB), jnp.int32),
            pltpu.VMEM((_NSUB, _ISUB), jnp.int32),
            pltpu.VMEM((_C, E), jnp.float32),
            pltpu.VMEM((_C, E), jnp.float32),
            pltpu.VMEM((1, E), jnp.float32),
            pltpu.SemaphoreType.DMA,
            pltpu.SemaphoreType.DMA,
        ],
    )
    def emb_kernel(inputs_hbm, tok_hbm, pos_hbm, out_hbm,
                   idx0, idx1, rows0, rows1, pos_v, sem0, sem1):
        wid = lax.axis_index("s") * NC + lax.axis_index("c")
        t0 = wid * tpw

        gather_start(t0, inputs_hbm, tok_hbm, idx0, rows0, sem0)

        def pair_body(i2, _):
            t = t0 + i2 * 2
            gather_start(t + 1, inputs_hbm, tok_hbm, idx1, rows1, sem1)
            process(t, tok_hbm, pos_hbm, out_hbm, idx0, rows0, pos_v, sem0)

            @pl.when(i2 * 2 + 2 < tpw)
            def _prefetch():
                gather_start(t + 2, inputs_hbm, tok_hbm, idx0, rows0, sem0)

            process(t + 1, tok_hbm, pos_hbm, out_hbm, idx1, rows1, pos_v, sem1)
            return _

        lax.fori_loop(0, tpw // 2, pair_body, 0)

    return emb_kernel


def kernel(inputs, tok_table, pos_table):
    L, B = inputs.shape
    V, E = tok_table.shape
    P = pos_table.shape[0]
    emb = _make_kernel(L, B, E, V, P)
    inputs2 = inputs.reshape(L * B // _ISUB, _ISUB)
    out = emb(inputs2, tok_table, pos_table)
    return out.reshape(L, B, E)


# speedup vs baseline: 2.7765x; 1.0065x over previous
import functools

import jax
import jax.numpy as jnp
from jax import lax
from jax.experimental import pallas as pl
from jax.experimental.pallas import tpu as pltpu
from jax.experimental.pallas import tpu_sc as plsc

_LANES = 16
_C = 512
_ISUB = 128
_NSUB = _C // _ISUB


def _make_kernel(L, B, E, V, P):
    N = L * B
    n_chunks = N // _C
    info = plsc.get_sparse_core_info()
    NC, NS = info.num_cores, info.num_subcores
    NW = NC * NS
    tpw = n_chunks // NW
    chunks_per_l = B // _C
    assert B % _C == 0 and N % (_C * NW) == 0 and tpw % 2 == 0
    assert E % _LANES == 0
    ev = E // _LANES
    scale = float(E) ** 0.5

    mesh = plsc.VectorSubcoreMesh(core_axis_name="c", subcore_axis_name="s")

    def gather_start(t, inputs_hbm, tok_hbm, idx_v, rows_v, sem):
        pltpu.sync_copy(inputs_hbm.at[pl.ds(t * _NSUB, _NSUB)], idx_v)
        for j in range(_NSUB):
            pltpu.async_copy(
                tok_hbm.at[idx_v.at[j]],
                rows_v.at[pl.ds(j * _ISUB, _ISUB)],
                sem,
            )

    def gather_wait(tok_hbm, idx_v, rows_v, sem):
        for j in range(_NSUB):
            pltpu.make_async_copy(
                tok_hbm.at[idx_v.at[j]],
                rows_v.at[pl.ds(j * _ISUB, _ISUB)],
                sem,
            ).wait()

    def process(t, tok_hbm, pos_hbm, out_hbm, idx_v, rows_v, pos_v, sem):
        l = t // chunks_per_l
        pltpu.sync_copy(pos_hbm.at[pl.ds(l, 1)], pos_v)
        pvs = [pos_v[0, pl.ds(j * _LANES, _LANES)] for j in range(ev)]
        gather_wait(tok_hbm, idx_v, rows_v, sem)

        @plsc.parallel_loop(0, _C, unroll=8)
        def _rows(r):
            for j in range(ev):
                sl = pl.ds(j * _LANES, _LANES)
                rows_v[r, sl] = rows_v[r, sl] * scale + pvs[j]
        pltpu.sync_copy(rows_v, out_hbm.at[pl.ds(t * _C, _C)])

    @functools.partial(
        pl.kernel,
        mesh=mesh,
        out_type=jax.ShapeDtypeStruct((N, E), jnp.float32),
        compiler_params=pltpu.CompilerParams(use_tc_tiling_on_sc=False),
        scratch_types=[
            pltpu.VMEM((_NSUB, _ISUB), jnp.int32),
            pltpu.VMEM((_NSUB, _ISUB), jnp.int32),
            pltpu.VMEM((_C, E), jnp.float32),
            pltpu.VMEM((_C, E), jnp.float32),
            pltpu.VMEM((1, E), jnp.float32),
            pltpu.SemaphoreType.DMA,
            pltpu.SemaphoreType.DMA,
        ],
    )
    def emb_kernel(inputs_hbm, tok_hbm, pos_hbm, out_hbm,
                   idx0, idx1, rows0, rows1, pos_v, sem0, sem1):
        wid = lax.axis_index("s") * NC + lax.axis_index("c")
        t0 = wid * tpw

        gather_start(t0, inputs_hbm, tok_hbm, idx0, rows0, sem0)

        def pair_body(i2, _):
            t = t0 + i2 * 2
            gather_start(t + 1, inputs_hbm, tok_hbm, idx1, rows1, sem1)
            process(t, tok_hbm, pos_hbm, out_hbm, idx0, rows0, pos_v, sem0)

            @pl.when(i2 * 2 + 2 < tpw)
            def _prefetch():
                gather_start(t + 2, inputs_hbm, tok_hbm, idx0, rows0, sem0)

            process(t + 1, tok_hbm, pos_hbm, out_hbm, idx1, rows1, pos_v, sem1)
            return _

        lax.fori_loop(0, tpw // 2, pair_body, 0)

    return emb_kernel


def kernel(inputs, tok_table, pos_table):
    L, B = inputs.shape
    V, E = tok_table.shape
    P = pos_table.shape[0]
    emb = _make_kernel(L, B, E, V, P)
    inputs2 = inputs.reshape(L * B // _ISUB, _ISUB)
    out = emb(inputs2, tok_table, pos_table)
    return out.reshape(L, B, E)


# speedup vs baseline: 3.4488x; 1.2422x over previous
import functools

import jax
import jax.numpy as jnp
from jax import lax
from jax.experimental import pallas as pl
from jax.experimental.pallas import tpu as pltpu
from jax.experimental.pallas import tpu_sc as plsc

_LANES = 16
_ROW = 128
_C = 256
_ISUB = 128
_NSUB = _C // _ISUB


def _make_kernel(L, B, E):
    N = L * B
    n_chunks = N // _C
    info = plsc.get_sparse_core_info()
    NC, NS = info.num_cores, info.num_subcores
    NW = NC * NS
    tpw = n_chunks // NW
    chunks_per_l = B // _C
    assert B % _C == 0 and N % (_C * NW) == 0 and tpw % 2 == 0
    assert E % _LANES == 0 and E <= _ROW
    ev = E // _LANES
    scale = float(E) ** 0.5

    mesh = plsc.VectorSubcoreMesh(core_axis_name="c", subcore_axis_name="s")

    @functools.partial(
        pl.kernel,
        mesh=mesh,
        out_type=jax.ShapeDtypeStruct((N, E), jnp.float32),
        scratch_types=[
            pltpu.VMEM((_NSUB, _ISUB), jnp.int32),
            pltpu.VMEM((_NSUB, _ISUB), jnp.int32),
            pltpu.VMEM((_C, _ROW), jnp.float32),
            pltpu.VMEM((_C, _ROW), jnp.float32),
            pltpu.VMEM((_C, E), jnp.float32),
            pltpu.VMEM((1, E), jnp.float32),
            pltpu.SemaphoreType.DMA,
            pltpu.SemaphoreType.DMA,
        ],
    )
    def emb_kernel(inputs_hbm, tok_hbm, pos_hbm, out_hbm,
                   idx0, idx1, rows0, rows1, out_v, pos_v, sem0, sem1):
        wid = lax.axis_index("s") * NC + lax.axis_index("c")
        t0 = wid * tpw

        def g_start(t, idx_v, rows_v, sem):
            pltpu.sync_copy(inputs_hbm.at[pl.ds(t * _NSUB, _NSUB)], idx_v)
            for j in range(_NSUB):
                pltpu.async_copy(
                    tok_hbm.at[idx_v.at[j]],
                    rows_v.at[pl.ds(j * _ISUB, _ISUB)],
                    sem,
                )

        def process(t, idx_v, rows_v, sem):
            l = t // chunks_per_l
            pltpu.sync_copy(pos_hbm.at[pl.ds(l, 1)], pos_v)
            pvs = [pos_v[0, pl.ds(j * _LANES, _LANES)] for j in range(ev)]
            for j in range(_NSUB):
                pltpu.make_async_copy(
                    tok_hbm.at[idx_v.at[j]],
                    rows_v.at[pl.ds(j * _ISUB, _ISUB)],
                    sem,
                ).wait()

            @plsc.parallel_loop(0, _C, unroll=8)
            def _rows(r):
                for j in range(ev):
                    sl = pl.ds(j * _LANES, _LANES)
                    out_v[r, sl] = rows_v[r, sl] * scale + pvs[j]

            pltpu.sync_copy(out_v, out_hbm.at[pl.ds(t * _C, _C)])

        g_start(t0, idx0, rows0, sem0)

        def pair_body(i2, _):
            t = t0 + i2 * 2
            g_start(t + 1, idx1, rows1, sem1)
            process(t, idx0, rows0, sem0)

            @pl.when(i2 * 2 + 2 < tpw)
            def _prefetch():
                g_start(t + 2, idx0, rows0, sem0)

            process(t + 1, idx1, rows1, sem1)
            return _

        lax.fori_loop(0, tpw // 2, pair_body, 0)

    return emb_kernel


def kernel(inputs, tok_table, pos_table):
    L, B = inputs.shape
    V, E = tok_table.shape
    emb = _make_kernel(L, B, E)
    tok_pad = jnp.pad(tok_table, ((0, 0), (0, _ROW - E)))
    inputs2 = inputs.reshape(L * B // _ISUB, _ISUB)
    out = emb(inputs2, tok_pad, pos_table)
    return out.reshape(L, B, E)
